# baseline (speedup 1.0000x reference)
import functools

import jax
import jax.numpy as jnp
from jax import lax
from jax.experimental import pallas as pl
from jax.experimental.pallas import tpu as pltpu
from jax.experimental.pallas import tpu_sc as plsc

B, N, NSYM = 16384, 200, 8
L = 16
NC, NS = 2, 16
NW = NC * NS
ROWS_PER_W = B // NW
CH = 64
NCHUNK = ROWS_PER_W // CH
CHW = CH * N
VPG = (2 * N) // L
GPC = CH // 2
SPLIT = N // L


def _sc_body(idx_hbm, c_hbm, w_hbm, out_hbm, wtab, ibuf, cbuf, obuf):
    wid = lax.axis_index("s") * NC + lax.axis_index("c")
    base = wid * (ROWS_PER_W * N)
    pltpu.sync_copy(w_hbm, wtab)
    wreg = wtab[...]

    lane = lax.iota(jnp.int32, 16)
    mlow = lane < (N - SPLIT * L)
    zf = jnp.zeros((L,), jnp.float32)

    def chunk_body(t, carry):
        off = base + t * CHW
        pltpu.sync_copy(idx_hbm.at[pl.ds(off, CHW)], ibuf)
        pltpu.sync_copy(c_hbm.at[pl.ds(off, CHW)], cbuf)

        def group_body(g, carry2):
            gb = g * (2 * N)
            cv = []
            wv = []
            acc0c = zf
            acc1c = zf
            acc0w = zf
            acc1w = zf
            for i in range(VPG):
                c = cbuf[pl.ds(gb + L * i, L)]
                ix = ibuf[pl.ds(gb + L * i, L)]
                w = jnp.take_along_axis(wreg, ix, axis=0)
                cv.append(c)
                wv.append(w)
                if i < SPLIT:
                    acc0c = acc0c + c
                    acc0w = acc0w + w
                elif i == SPLIT:
                    acc0c = acc0c + jnp.where(mlow, c, zf)
                    acc0w = acc0w + jnp.where(mlow, w, zf)
                    acc1c = acc1c + jnp.where(mlow, zf, c)
                    acc1w = acc1w + jnp.where(mlow, zf, w)
                else:
                    acc1c = acc1c + c
                    acc1w = acc1w + w
            v0 = jnp.broadcast_to(0.0 - jnp.sum(acc0c), (L,)) / jnp.broadcast_to(
                jnp.sum(acc0w), (L,)
            )
            v1 = jnp.broadcast_to(0.0 - jnp.sum(acc1c), (L,)) / jnp.broadcast_to(
                jnp.sum(acc1w), (L,)
            )
            vm = jnp.where(mlow, v0, v1)
            for i in range(VPG):
                sc = v0 if i < SPLIT else (vm if i == SPLIT else v1)
                obuf[pl.ds(gb + L * i, L)] = cv[i] + sc * wv[i]
            return carry2

        lax.fori_loop(0, GPC, group_body, 0)
        pltpu.sync_copy(obuf, out_hbm.at[pl.ds(off, CHW)])
        return carry

    lax.fori_loop(0, NCHUNK, chunk_body, 0)


def kernel(element_idxs, raw_charges, weights):
    idx_flat = element_idxs.reshape(B * N)
    c_flat = raw_charges.reshape(B * N)
    w16 = jnp.concatenate(
        [weights.astype(jnp.float32), jnp.zeros((L - NSYM,), jnp.float32)]
    )
    mesh = plsc.VectorSubcoreMesh(core_axis_name="c", subcore_axis_name="s")
    f = pl.kernel(
        _sc_body,
        mesh=mesh,
        compiler_params=pltpu.CompilerParams(needs_layout_passes=False),
        out_type=jax.ShapeDtypeStruct((B * N,), jnp.float32),
        scratch_types=[
            pltpu.VMEM((L,), jnp.float32),
            pltpu.VMEM((CHW,), jnp.int32),
            pltpu.VMEM((CHW,), jnp.float32),
            pltpu.VMEM((CHW,), jnp.float32),
        ],
    )
    out = f(idx_flat, c_flat, w16)
    return out.reshape(B, N)


# speedup vs baseline: 1.7024x; 1.7024x over previous
import functools

import jax
import jax.numpy as jnp
from jax import lax
from jax.experimental import pallas as pl
from jax.experimental.pallas import tpu as pltpu
from jax.experimental.pallas import tpu_sc as plsc

B, N, NSYM = 16384, 200, 8
L = 16
NC, NS = 2, 16
NW = NC * NS
ROWS_PER_W = B // NW
CH = 64
NCHUNK = ROWS_PER_W // CH
NFULL = N // L
TAIL = N - L


def _row_sums(cbuf, ibuf, b, row, wreg, mhigh, zf):
    cv = [cbuf[b, row, pl.ds(L * i, L)] for i in range(NFULL)]
    iv = [ibuf[b, row, pl.ds(L * i, L)] for i in range(NFULL)]
    ct = cbuf[b, row, pl.ds(TAIL, L)]
    it = ibuf[b, row, pl.ds(TAIL, L)]
    wv = [jnp.take_along_axis(wreg, ix, axis=0) for ix in iv]
    wt = jnp.take_along_axis(wreg, it, axis=0)
    sc_ = _tree(cv)
    sw_ = _tree(wv)
    sc_ = sc_ + jnp.where(mhigh, ct, zf)
    sw_ = sw_ + jnp.where(mhigh, wt, zf)
    scale = jnp.broadcast_to(0.0 - jnp.sum(sc_), (L,)) / jnp.broadcast_to(
        jnp.sum(sw_), (L,)
    )
    return cv + [ct], wv + [wt], scale


def _tree(vs):
    vs = list(vs)
    while len(vs) > 1:
        nxt = [vs[i] + vs[i + 1] for i in range(0, len(vs) - 1, 2)]
        if len(vs) % 2:
            nxt.append(vs[-1])
        vs = nxt
    return vs[0]


def _sc_body(idx_hbm, c_hbm, w_hbm, out_hbm, wtab, ibuf, cbuf, obuf,
             si0, si1, sc0, sc1):
    wid = lax.axis_index("s") * NC + lax.axis_index("c")
    row0 = wid * ROWS_PER_W
    pltpu.sync_copy(w_hbm, wtab.at[pl.ds(0, NSYM)])
    wreg = wtab[...]

    lane = lax.iota(jnp.int32, 16)
    mhigh = lane >= (L - (N - L * NFULL))
    zf = jnp.zeros((L,), jnp.float32)
    sem_i = (si0, si1)
    sem_c = (sc0, sc1)

    def start_in(k):
        b = k % 2
        cp_i = pltpu.async_copy(
            idx_hbm.at[pl.ds(row0 + k * CH, CH)], ibuf.at[b], sem_i[b]
        )
        cp_c = pltpu.async_copy(
            c_hbm.at[pl.ds(row0 + k * CH, CH)], cbuf.at[b], sem_c[b]
        )
        return cp_i, cp_c

    pending = {0: start_in(0)}
    for k in range(NCHUNK):
        b = k % 2
        if k + 1 < NCHUNK:
            pending[k + 1] = start_in(k + 1)
        cp_i, cp_c = pending.pop(k)
        cp_i.wait()
        cp_c.wait()

        def pair_body(g, carry, b=b):
            r = 2 * g
            cv0, wv0, s0 = _row_sums(cbuf, ibuf, b, r, wreg, mhigh, zf)
            cv1, wv1, s1 = _row_sums(cbuf, ibuf, b, r + 1, wreg, mhigh, zf)
            for i in range(NFULL):
                obuf[r, pl.ds(L * i, L)] = cv0[i] + s0 * wv0[i]
                obuf[r + 1, pl.ds(L * i, L)] = cv1[i] + s1 * wv1[i]
            obuf[r, pl.ds(TAIL, L)] = cv0[NFULL] + s0 * wv0[NFULL]
            obuf[r + 1, pl.ds(TAIL, L)] = cv1[NFULL] + s1 * wv1[NFULL]
            return carry

        lax.fori_loop(0, CH // 2, pair_body, 0)
        pltpu.sync_copy(obuf, out_hbm.at[pl.ds(row0 + k * CH, CH)])


def kernel(element_idxs, raw_charges, weights):
    mesh = plsc.VectorSubcoreMesh(core_axis_name="c", subcore_axis_name="s")
    f = pl.kernel(
        _sc_body,
        mesh=mesh,
        compiler_params=pltpu.CompilerParams(needs_layout_passes=False),
        out_type=jax.ShapeDtypeStruct((B, N), jnp.float32),
        scratch_types=[
            pltpu.VMEM((L,), jnp.float32),
            pltpu.VMEM((2, CH, N), jnp.int32),
            pltpu.VMEM((2, CH, N), jnp.float32),
            pltpu.VMEM((CH, N), jnp.float32),
            pltpu.SemaphoreType.DMA,
            pltpu.SemaphoreType.DMA,
            pltpu.SemaphoreType.DMA,
            pltpu.SemaphoreType.DMA,
        ],
    )
    return f(element_idxs, raw_charges, weights)


# speedup vs baseline: 1.7034x; 1.0006x over previous
import functools

import jax
import jax.numpy as jnp
from jax import lax
from jax.experimental import pallas as pl
from jax.experimental.pallas import tpu as pltpu
from jax.experimental.pallas import tpu_sc as plsc

B, N, NSYM = 16384, 200, 8
L = 16
NC, NS = 2, 16
NW = NC * NS
ROWS_PER_W = B // NW
CH = 64
NCHUNK = ROWS_PER_W // CH
NFULL = N // L
TAIL = N - L


def _row_sums(cbuf, ibuf, b, row, wreg, mhigh, zf):
    cv = [cbuf[b, row, pl.ds(L * i, L)] for i in range(NFULL)]
    iv = [ibuf[b, row, pl.ds(L * i, L)] for i in range(NFULL)]
    ct = cbuf[b, row, pl.ds(TAIL, L)]
    it = ibuf[b, row, pl.ds(TAIL, L)]
    wv = [jnp.take_along_axis(wreg, ix, axis=0) for ix in iv]
    wt = jnp.take_along_axis(wreg, it, axis=0)
    sc_ = _tree(cv)
    sw_ = _tree(wv)
    sc_ = sc_ + jnp.where(mhigh, ct, zf)
    sw_ = sw_ + jnp.where(mhigh, wt, zf)
    scale = jnp.broadcast_to(0.0 - jnp.sum(sc_), (L,)) / jnp.broadcast_to(
        jnp.sum(sw_), (L,)
    )
    return cv + [ct], wv + [wt], scale


def _tree(vs):
    vs = list(vs)
    while len(vs) > 1:
        nxt = [vs[i] + vs[i + 1] for i in range(0, len(vs) - 1, 2)]
        if len(vs) % 2:
            nxt.append(vs[-1])
        vs = nxt
    return vs[0]


def _sc_body(idx_hbm, c_hbm, w_hbm, out_hbm, wtab, ibuf, cbuf, obuf,
             si0, si1, sc0, sc1):
    wid = lax.axis_index("s") * NC + lax.axis_index("c")
    row0 = wid * ROWS_PER_W
    pltpu.sync_copy(w_hbm, wtab.at[pl.ds(0, NSYM)])
    wreg = wtab[...]

    lane = lax.iota(jnp.int32, 16)
    mhigh = lane >= (L - (N - L * NFULL))
    zf = jnp.zeros((L,), jnp.float32)
    sem_i = (si0, si1)
    sem_c = (sc0, sc1)

    def start_in(k):
        b = k % 2
        cp_i = pltpu.async_copy(
            idx_hbm.at[pl.ds(row0 + k * CH, CH)], ibuf.at[b], sem_i[b]
        )
        cp_c = pltpu.async_copy(
            c_hbm.at[pl.ds(row0 + k * CH, CH)], cbuf.at[b], sem_c[b]
        )
        return cp_i, cp_c

    pending = {0: start_in(0)}
    for k in range(NCHUNK):
        b = k % 2
        if k + 1 < NCHUNK:
            pending[k + 1] = start_in(k + 1)
        cp_i, cp_c = pending.pop(k)
        cp_i.wait()
        cp_c.wait()

        def pair_body(g, carry, b=b):
            r = 2 * g
            cv0, wv0, s0 = _row_sums(cbuf, ibuf, b, r, wreg, mhigh, zf)
            cv1, wv1, s1 = _row_sums(cbuf, ibuf, b, r + 1, wreg, mhigh, zf)
            for i in range(NFULL):
                obuf[r, pl.ds(L * i, L)] = cv0[i] + s0 * wv0[i]
                obuf[r + 1, pl.ds(L * i, L)] = cv1[i] + s1 * wv1[i]
            obuf[r, pl.ds(TAIL, L)] = cv0[NFULL] + s0 * wv0[NFULL]
            obuf[r + 1, pl.ds(TAIL, L)] = cv1[NFULL] + s1 * wv1[NFULL]
            return carry

        lax.fori_loop(0, CH // 2, pair_body, 0)
        pltpu.sync_copy(obuf, out_hbm.at[pl.ds(row0 + k * CH, CH)])


def kernel(element_idxs, raw_charges, weights):
    mesh = plsc.VectorSubcoreMesh(core_axis_name="c", subcore_axis_name="s")
    f = pl.kernel(
        _sc_body,
        mesh=mesh,
        compiler_params=pltpu.CompilerParams(
            needs_layout_passes=False, use_tc_tiling_on_sc=True
        ),
        out_type=jax.ShapeDtypeStruct((B, N), jnp.float32),
        scratch_types=[
            pltpu.VMEM((L,), jnp.float32),
            pltpu.VMEM((2, CH, N), jnp.int32),
            pltpu.VMEM((2, CH, N), jnp.float32),
            pltpu.VMEM((CH, N), jnp.float32),
            pltpu.SemaphoreType.DMA,
            pltpu.SemaphoreType.DMA,
            pltpu.SemaphoreType.DMA,
            pltpu.SemaphoreType.DMA,
        ],
    )
    return f(element_idxs, raw_charges, weights)


# speedup vs baseline: 3.2476x; 1.9066x over previous
import functools

import jax
import jax.numpy as jnp
from jax import lax
from jax.experimental import pallas as pl
from jax.experimental.pallas import tpu as pltpu
from jax.experimental.pallas import tpu_sc as plsc

B, N, NSYM = 16384, 200, 8
L = 16
NC, NS = 2, 16
NW = NC * NS
MOLS_PER_W = B // NW
MB = 128
NCHUNK = MOLS_PER_W // MB
NG = MB // L


def _sc_body(idx_hbm, c_hbm, w_hbm, out_hbm, wtab, ibuf, cbuf,
             si0, si1, sc0, sc1, so0, so1):
    wid = lax.axis_index("s") * NC + lax.axis_index("c")
    m0 = wid * MOLS_PER_W
    pltpu.sync_copy(w_hbm, wtab.at[pl.ds(0, NSYM)])
    wreg = wtab[...]

    zf = jnp.zeros((L,), jnp.float32)
    sem_i = (si0, si1)
    sem_c = (sc0, sc1)
    sem_o = (so0, so1)

    def col(k):
        return pl.ds(m0 + k * MB, MB)

    def start_in(k):
        b = k % 2
        return (
            pltpu.async_copy(idx_hbm.at[:, col(k)], ibuf.at[b], sem_i[b]),
            pltpu.async_copy(c_hbm.at[:, col(k)], cbuf.at[b], sem_c[b]),
        )

    pending_in = {0: start_in(0)}
    pending_out = {}
    for k in range(NCHUNK):
        b = k % 2
        if k + 1 < NCHUNK:
            if k - 1 in pending_out:
                pending_out.pop(k - 1).wait()
            pending_in[k + 1] = start_in(k + 1)
        cp_i, cp_c = pending_in.pop(k)
        cp_i.wait()
        cp_c.wait()

        def sum_body(n, accs, b=b):
            accs = list(accs)
            for g in range(NG):
                c = cbuf[b, n, pl.ds(L * g, L)]
                ix = plsc.bitcast(ibuf[b, n, pl.ds(L * g, L)], jnp.int32)
                w = jnp.take_along_axis(wreg, ix, axis=0)
                ac, aw = accs[g]
                accs[g] = (ac + c, aw + w)
            return tuple(accs)

        accs = lax.fori_loop(0, N, sum_body, tuple((zf, zf) for _ in range(NG)))
        scales = [(0.0 - ac) / aw for ac, aw in accs]

        def out_body(n, carry, b=b, scales=scales):
            for g in range(NG):
                c = cbuf[b, n, pl.ds(L * g, L)]
                ix = plsc.bitcast(ibuf[b, n, pl.ds(L * g, L)], jnp.int32)
                w = jnp.take_along_axis(wreg, ix, axis=0)
                ibuf[b, n, pl.ds(L * g, L)] = c + scales[g] * w
            return carry

        lax.fori_loop(0, N, out_body, 0)
        pending_out[k] = pltpu.async_copy(
            ibuf.at[b], out_hbm.at[:, col(k)], sem_o[b]
        )

    for k in sorted(pending_out):
        pending_out[k].wait()


def kernel(element_idxs, raw_charges, weights):
    mesh = plsc.VectorSubcoreMesh(core_axis_name="c", subcore_axis_name="s")
    f = pl.kernel(
        _sc_body,
        mesh=mesh,
        compiler_params=pltpu.CompilerParams(needs_layout_passes=False),
        out_type=jax.ShapeDtypeStruct((N, B), jnp.float32),
        scratch_types=[
            pltpu.VMEM((L,), jnp.float32),
            pltpu.VMEM((2, N, MB), jnp.float32),
            pltpu.VMEM((2, N, MB), jnp.float32),
            pltpu.SemaphoreType.DMA,
            pltpu.SemaphoreType.DMA,
            pltpu.SemaphoreType.DMA,
            pltpu.SemaphoreType.DMA,
            pltpu.SemaphoreType.DMA,
            pltpu.SemaphoreType.DMA,
        ],
    )
    out_t = f(element_idxs.T.view(jnp.float32), raw_charges.T, weights)
    return out_t.T
